# baseline (speedup 1.0000x reference)
import jax
import jax.numpy as jnp
from jax import lax
from jax.experimental import pallas as pl
from jax.experimental.pallas import tpu as pltpu
from jax.experimental.pallas import tpu_sc as plsc

_HC = 1024
_WC = 1024
_R = 32
_SROWS = _R + 8
_LANES = 16
_NSTAGE = 4
_NCH = _WC // _LANES


def _sc_body(patches, hw, lut, out, locv, lutv, stages, rowbuf, outstage,
             cntrow, recrow, sems, sem_out):
    B, N, C, Hp, Wp = patches.shape
    cid = lax.axis_index("c")
    sid = lax.axis_index("s")
    wid = sid * 2 + cid
    b = wid // C
    ch = wid % C

    pltpu.sync_copy(hw, locv)
    pltpu.sync_copy(lut, lutv)
    lane = lax.iota(jnp.int32, _LANES)
    hvec = locv[0, :]
    wvec = locv[1, :]
    zeros16 = jnp.zeros((_LANES,), jnp.float32)
    ones16 = jnp.ones((_LANES,), jnp.float32)

    @plsc.parallel_loop(0, _R * _NCH // 8, unroll=2)
    def _(t):
        base = pl.multiple_of(t * (_LANES * 8), _LANES)
        for u in range(8):
            rowbuf[pl.ds(base + _LANES * u, _LANES)] = zeros16

    def window(i, row0):
        lo = jnp.maximum(hvec[i], row0)
        hi = jnp.minimum(hvec[i] + Hp, row0 + _R)
        src_off = lo - hvec[i]
        src0 = jnp.minimum((src_off // 8) * 8, Hp - _SROWS)
        return lo, hi, src0

    def issue(i, row0):
        lo, hi, src0 = window(i, row0)

        @pl.when(hi > lo)
        def _():
            pltpu.async_copy(
                patches.at[b, i, ch, pl.ds(src0, _SROWS), :],
                stages[i % _NSTAGE],
                sems[i % _NSTAGE],
            )

    for i in range(_NSTAGE - 1):
        issue(i, 0)

    def block_body(blk, carry):
        row0 = blk * _R

        def accumulate(i):
            lo, hi, src0 = window(i, row0)

            @pl.when(hi > lo)
            def _():
                pltpu.make_async_copy(
                    patches.at[b, i, ch, pl.ds(src0, _SROWS), :],
                    stages[i % _NSTAGE],
                    sems[i % _NSTAGE],
                ).wait()
                k0 = lo - hvec[i] - src0
                w_i = wvec[i]
                stg = stages[i % _NSTAGE]

                @plsc.parallel_loop(0, hi - lo, unroll=2)
                def _(k):
                    base = (lo - row0 + k) * _WC + w_i
                    ks = k0 + k
                    for j in range(Wp // _LANES):
                        v = stg[ks, pl.ds(_LANES * j, _LANES)]
                        idx = (base + _LANES * j) + lane
                        plsc.addupdate_scatter(rowbuf, [idx], v)

        for i in range(N):
            if i + _NSTAGE - 1 < N:
                issue(i + _NSTAGE - 1, row0)
            accumulate(i)
        for i in range(_NSTAGE - 1):
            issue(i, row0 + _R)

        @pl.when(blk > 0)
        def _():
            pltpu.make_async_copy(
                outstage, out.at[b, ch, pl.ds((blk - 1) * _R, _R), :], sem_out
            ).wait()

        def seg_cond(seg_start):
            return seg_start < row0 + _R

        def seg_body(seg_start):
            seg_end = row0 + _R
            for i in range(N):
                for bound in (hvec[i], hvec[i] + Hp):
                    take = (bound > seg_start) & (bound < seg_end)
                    seg_end = jnp.where(take, bound, seg_end)

            @plsc.parallel_loop(0, _NCH, unroll=4)
            def _(j):
                joff = pl.multiple_of(_LANES * j, _LANES)
                cntrow[pl.ds(joff, _LANES)] = zeros16

            for i in range(N):
                h_i = hvec[i]
                w_i = wvec[i]

                @pl.when((h_i <= seg_start) & (seg_start < h_i + Hp))
                def _():
                    for j in range(Wp // _LANES):
                        idx = (w_i + _LANES * j) + lane
                        plsc.addupdate_scatter(cntrow, [idx], ones16)

            @plsc.parallel_loop(0, _NCH, unroll=4)
            def _(j):
                joff = pl.multiple_of(_LANES * j, _LANES)
                cnt = cntrow[pl.ds(joff, _LANES)]
                recrow[pl.ds(joff, _LANES)] = plsc.load_gather(
                    lutv, [cnt.astype(jnp.int32)]
                )

            r0 = seg_start - row0

            @plsc.parallel_loop(0, seg_end - seg_start)
            def _(k):
                r = r0 + k
                rb = r * _WC
                for j in range(_NCH):
                    joff = pl.multiple_of(rb + _LANES * j, _LANES)
                    outstage[r, pl.ds(_LANES * j, _LANES)] = (
                        rowbuf[pl.ds(joff, _LANES)]
                        * recrow[pl.ds(_LANES * j, _LANES)]
                    )
                    rowbuf[pl.ds(joff, _LANES)] = zeros16

            return seg_end

        lax.while_loop(seg_cond, seg_body, row0)

        pltpu.async_copy(outstage, out.at[b, ch, pl.ds(row0, _R), :], sem_out)
        return carry

    nblk = _HC // _R
    lax.fori_loop(0, nblk, block_body, 0)
    pltpu.make_async_copy(
        outstage, out.at[b, ch, pl.ds((nblk - 1) * _R, _R), :], sem_out
    ).wait()


def kernel(patches, locations, H, W):
    B, N, C, Hp, Wp = patches.shape
    hs = jnp.minimum(locations[:, 0], _HC - Hp).astype(jnp.int32)
    ws = jnp.minimum(locations[:, 1], _WC - Wp).astype(jnp.int32)
    hw = jnp.stack([hs, ws])
    lut = 1.0 / (jnp.arange(32, dtype=jnp.float32) + 1e-8)

    mesh = plsc.VectorSubcoreMesh(core_axis_name="c", subcore_axis_name="s")
    fn = pl.kernel(
        _sc_body,
        out_type=jax.ShapeDtypeStruct((B, C, _HC, _WC), jnp.float32),
        mesh=mesh,
        compiler_params=pltpu.CompilerParams(needs_layout_passes=False),
        scratch_types=[
            pltpu.VMEM((2, N), jnp.int32),
            pltpu.VMEM((32,), jnp.float32),
            [pltpu.VMEM((_SROWS, Wp), jnp.float32) for _ in range(_NSTAGE)],
            pltpu.VMEM((_R * _WC,), jnp.float32),
            pltpu.VMEM((_R, _WC), jnp.float32),
            pltpu.VMEM((_WC,), jnp.float32),
            pltpu.VMEM((_WC,), jnp.float32),
            [pltpu.SemaphoreType.DMA for _ in range(_NSTAGE)],
            pltpu.SemaphoreType.DMA,
        ],
    )
    return fn(patches, hw, lut)


# speedup vs baseline: 1.1593x; 1.1593x over previous
import jax
import jax.numpy as jnp
from jax import lax
from jax.experimental import pallas as pl
from jax.experimental.pallas import tpu as pltpu
from jax.experimental.pallas import tpu_sc as plsc

_HC = 1024
_WC = 1024
_R = 32
_SROWS = _R + 8
_LANES = 16
_NSTAGE = 4
_NCH = _WC // _LANES


def _sc_body(patches, hw, lut, out, locv, lutv, stages, rowbuf, outstage,
             cntrow, recrow, sems, sem_out):
    B, N, C, Hp, Wp = patches.shape
    cid = lax.axis_index("c")
    sid = lax.axis_index("s")
    wid = sid * 2 + cid
    b = wid // C
    ch = wid % C

    pltpu.sync_copy(hw, locv)
    pltpu.sync_copy(lut, lutv)
    lane = lax.iota(jnp.int32, _LANES)
    hvec = locv[0, :]
    wvec = locv[1, :]
    zeros16 = jnp.zeros((_LANES,), jnp.float32)
    ones16 = jnp.ones((_LANES,), jnp.float32)

    @plsc.parallel_loop(0, _R * _NCH // 8, unroll=2)
    def _(t):
        base = pl.multiple_of(t * (_LANES * 8), _LANES)
        for u in range(8):
            rowbuf[pl.ds(base + _LANES * u, _LANES)] = zeros16

    def window(i, row0):
        lo = jnp.maximum(hvec[i], row0)
        hi = jnp.minimum(hvec[i] + Hp, row0 + _R)
        src_off = lo - hvec[i]
        src0 = jnp.minimum((src_off // 8) * 8, Hp - _SROWS)
        return lo, hi, src0

    def issue(i, row0):
        lo, hi, src0 = window(i, row0)

        @pl.when(hi > lo)
        def _():
            pltpu.async_copy(
                patches.at[b, i, ch, pl.ds(src0, _SROWS), :],
                stages[i % _NSTAGE],
                sems[i % _NSTAGE],
            )

    for i in range(_NSTAGE - 1):
        issue(i, 0)

    def block_body(blk, carry):
        row0 = blk * _R

        def accumulate(i):
            lo, hi, src0 = window(i, row0)

            @pl.when(hi > lo)
            def _():
                pltpu.make_async_copy(
                    patches.at[b, i, ch, pl.ds(src0, _SROWS), :],
                    stages[i % _NSTAGE],
                    sems[i % _NSTAGE],
                ).wait()
                k0 = lo - hvec[i] - src0
                w_i = wvec[i]
                stg = stages[i % _NSTAGE]

                @plsc.parallel_loop(0, hi - lo, unroll=2)
                def _(k):
                    base = (lo - row0 + k) * _WC + w_i
                    ks = k0 + k
                    for j in range(Wp // _LANES):
                        v = stg[ks, pl.ds(_LANES * j, _LANES)]
                        idx = (base + _LANES * j) + lane
                        plsc.addupdate_scatter(rowbuf, [idx], v)

        for i in range(N):
            if i + _NSTAGE - 1 < N:
                issue(i + _NSTAGE - 1, row0)
            accumulate(i)
        for i in range(_NSTAGE - 1):
            issue(i, row0 + _R)

        @pl.when(blk > 0)
        def _():
            pltpu.make_async_copy(
                outstage, out.at[b, ch, pl.ds((blk - 1) * _R, _R), :], sem_out
            ).wait()

        def seg_cond(seg_start):
            return seg_start < row0 + _R

        def seg_body(seg_start):
            seg_end = row0 + _R
            for i in range(N):
                for bound in (hvec[i], hvec[i] + Hp):
                    take = (bound > seg_start) & (bound < seg_end)
                    seg_end = jnp.where(take, bound, seg_end)

            @plsc.parallel_loop(0, _NCH, unroll=4)
            def _(j):
                joff = pl.multiple_of(_LANES * j, _LANES)
                cntrow[pl.ds(joff, _LANES)] = zeros16

            for i in range(N):
                h_i = hvec[i]
                w_i = wvec[i]

                @pl.when((h_i <= seg_start) & (seg_start < h_i + Hp))
                def _():
                    for j in range(Wp // _LANES):
                        idx = (w_i + _LANES * j) + lane
                        plsc.addupdate_scatter(cntrow, [idx], ones16)

            @plsc.parallel_loop(0, _NCH, unroll=4)
            def _(j):
                joff = pl.multiple_of(_LANES * j, _LANES)
                cnt = cntrow[pl.ds(joff, _LANES)]
                recrow[pl.ds(joff, _LANES)] = plsc.load_gather(
                    lutv, [cnt.astype(jnp.int32)]
                )

            r0 = seg_start - row0

            @plsc.parallel_loop(0, (seg_end - seg_start) * _NCH, unroll=4)
            def _(t):
                r = r0 + t // _NCH
                joff = pl.multiple_of(_LANES * (t % _NCH), _LANES)
                rb = r * _WC + joff
                outstage[r, pl.ds(joff, _LANES)] = (
                    rowbuf[pl.ds(rb, _LANES)] * recrow[pl.ds(joff, _LANES)]
                )
                rowbuf[pl.ds(rb, _LANES)] = zeros16

            return seg_end

        lax.while_loop(seg_cond, seg_body, row0)

        pltpu.async_copy(outstage, out.at[b, ch, pl.ds(row0, _R), :], sem_out)
        return carry

    nblk = _HC // _R
    lax.fori_loop(0, nblk, block_body, 0)
    pltpu.make_async_copy(
        outstage, out.at[b, ch, pl.ds((nblk - 1) * _R, _R), :], sem_out
    ).wait()


def kernel(patches, locations, H, W):
    B, N, C, Hp, Wp = patches.shape
    hs = jnp.minimum(locations[:, 0], _HC - Hp).astype(jnp.int32)
    ws = jnp.minimum(locations[:, 1], _WC - Wp).astype(jnp.int32)
    hw = jnp.stack([hs, ws])
    lut = 1.0 / (jnp.arange(32, dtype=jnp.float32) + 1e-8)

    mesh = plsc.VectorSubcoreMesh(core_axis_name="c", subcore_axis_name="s")
    fn = pl.kernel(
        _sc_body,
        out_type=jax.ShapeDtypeStruct((B, C, _HC, _WC), jnp.float32),
        mesh=mesh,
        compiler_params=pltpu.CompilerParams(needs_layout_passes=False),
        scratch_types=[
            pltpu.VMEM((2, N), jnp.int32),
            pltpu.VMEM((32,), jnp.float32),
            [pltpu.VMEM((_SROWS, Wp), jnp.float32) for _ in range(_NSTAGE)],
            pltpu.VMEM((_R * _WC,), jnp.float32),
            pltpu.VMEM((_R, _WC), jnp.float32),
            pltpu.VMEM((_WC,), jnp.float32),
            pltpu.VMEM((_WC,), jnp.float32),
            [pltpu.SemaphoreType.DMA for _ in range(_NSTAGE)],
            pltpu.SemaphoreType.DMA,
        ],
    )
    return fn(patches, hw, lut)
